# baseline (speedup 1.0000x reference)
import jax
import jax.numpy as jnp
from jax.experimental import pallas as pl
from jax.experimental.pallas import tpu as pltpu

B = 8
H = W = 512
D = 64
NC = 1024
H1 = W1 = 256
HQ = WQ = 128
N = B * HQ * WQ
CHUNK = 1024
NCHUNKS = N // CHUNK

_f32 = jnp.float32


def _shift_axis(y, axis, d):
    if d == 0:
        return y
    zero_shape = list(y.shape)
    zero_shape[axis] = 1
    z = jnp.zeros(zero_shape, y.dtype)
    sl = [slice(None)] * y.ndim
    if d == -1:
        sl[axis] = slice(0, y.shape[axis] - 1)
        return jnp.concatenate([z, y[tuple(sl)]], axis=axis)
    else:
        sl[axis] = slice(1, None)
        return jnp.concatenate([y[tuple(sl)], z], axis=axis)


def _enc1_body(p_ref, w_ref, b_ref, o_ref):
    p = p_ref[0]
    acc = jnp.dot(w_ref[...], p, preferred_element_type=_f32)
    o_ref[0] = jnp.maximum(acc + b_ref[...], 0.0)


def _enc2_body(te_ref, to_ref, w_ref, b_ref, o_ref):
    te = te_ref[0]
    to = to_ref[0]
    hw = HQ * WQ

    def m(k, y, dy):
        return jnp.dot(w_ref[k], _shift_axis(y, 1, dy).reshape(128, hw),
                       preferred_element_type=_f32)

    acc = m(0, to, -1) + m(1, te, 0) + m(2, to, 0) + m(3, te, 1)
    o_ref[0] = jnp.maximum(acc + b_ref[...], 0.0)


def _vq_body(z_ref, cb_ref, cbt_ref, idx_ref, zq_ref, loss_ref):
    i = pl.program_id(0)
    z = z_ref[...]
    cb = cb_ref[...]
    s = jnp.dot(cb, z, preferred_element_type=_f32)
    cbn = jnp.sum(cb * cb, axis=1, keepdims=True)
    t = cbn - 2.0 * s
    m = jnp.min(t, axis=0, keepdims=True)
    rows = jax.lax.broadcasted_iota(jnp.int32, t.shape, 0)
    idx = jnp.min(jnp.where(t == m, rows, NC), axis=0, keepdims=True)
    idx_ref[0] = idx
    oh = (rows == idx).astype(_f32)
    zq_ref[...] = jnp.dot(cbt_ref[...], oh, preferred_element_type=_f32)
    part = jnp.sum(z * z, keepdims=True) + jnp.sum(m, keepdims=True)

    @pl.when(i == 0)
    def _init():
        loss_ref[...] = jnp.zeros_like(loss_ref)

    loss_ref[...] += part

    @pl.when(i == NCHUNKS - 1)
    def _norm():
        loss_ref[...] *= 1.0 / (N * D)


_PTAPS = {0: ((1, 0), (3, -1)), 1: ((0, 1), (2, 0))}


def _dec1_body(zq_ref, w_ref, b_ref, o_ref):
    zq = zq_ref[0].reshape(D, HQ, WQ)
    hw = HQ * WQ
    shifted = {}
    for dy in (-1, 0, 1):
        for dx in (-1, 0, 1):
            shifted[(dy, dx)] = _shift_axis(
                _shift_axis(zq, 1, dy), 2, dx).reshape(D, hw)
    for py in range(2):
        for px in range(2):
            acc = jnp.zeros((32, hw), _f32)
            for (kh, dy) in _PTAPS[py]:
                for (kw, dx) in _PTAPS[px]:
                    acc += jnp.dot(w_ref[kh * 4 + kw], shifted[(dy, dx)],
                                   preferred_element_type=_f32)
            o_ref[0, py * 2 + px] = jnp.maximum(acc + b_ref[...], 0.0)


def _dec2_body(d_ref, w_ref, b_ref, o_ref):
    d = d_ref[0]
    q = jnp.dot(w_ref[...], d, preferred_element_type=_f32)
    q = q.reshape(16, H1, W1)
    for py in range(2):
        for px in range(2):
            acc = jnp.zeros((H1, W1), _f32)
            for (kh, dy) in _PTAPS[py]:
                for (kw, dx) in _PTAPS[px]:
                    acc += _shift_axis(_shift_axis(q[kh * 4 + kw], 0, dy),
                                       1, dx)
            o_ref[0, py * 2 + px] = jax.nn.sigmoid(acc + b_ref[...])


def kernel(x, enc_w1, enc_b1, enc_w2, enc_b2, codebook,
           dec_w1, dec_b1, dec_w2, dec_b2):
    f = _f32
    xs = x[:, 0]
    xp = jnp.pad(xs, ((0, 0), (1, 1), (1, 1)))
    P = jnp.stack([xp[:, kh:kh + 511:2, kw:kw + 511:2].reshape(B, H1 * W1)
                   for kh in range(4) for kw in range(4)], axis=1)
    w1 = enc_w1.reshape(32, 16)
    b1 = enc_b1.reshape(32, 1)
    z1 = pl.pallas_call(
        _enc1_body,
        grid=(B,),
        in_specs=[
            pl.BlockSpec((1, 16, H1 * W1), lambda b: (b, 0, 0)),
            pl.BlockSpec((32, 16), lambda b: (0, 0)),
            pl.BlockSpec((32, 1), lambda b: (0, 0)),
        ],
        out_specs=pl.BlockSpec((1, 32, H1 * W1), lambda b: (b, 0, 0)),
        out_shape=jax.ShapeDtypeStruct((B, 32, H1 * W1), f),
    )(P, w1, b1)
    z1 = z1.reshape(B, 32, H1, W1)

    E, O = z1[:, :, 0::2, :], z1[:, :, 1::2, :]

    def col_taps(R):
        A, Bc = R[..., 0::2], R[..., 1::2]
        Bm = jnp.pad(Bc, ((0, 0), (0, 0), (0, 0), (1, 0)))[..., :WQ]
        Ap = jnp.pad(A, ((0, 0), (0, 0), (0, 0), (0, 1)))[..., 1:]
        return jnp.concatenate([Bm, A, Bc, Ap], axis=1)

    TE, TO = col_taps(E), col_taps(O)
    w2 = jnp.stack([jnp.concatenate([enc_w2[:, :, k, kw]
                                     for kw in range(4)], axis=1)
                    for k in range(4)])
    b2 = enc_b2.reshape(D, 1)
    z_e = pl.pallas_call(
        _enc2_body,
        grid=(B,),
        in_specs=[
            pl.BlockSpec((1, 128, HQ, WQ), lambda b: (b, 0, 0, 0)),
            pl.BlockSpec((1, 128, HQ, WQ), lambda b: (b, 0, 0, 0)),
            pl.BlockSpec((4, D, 128), lambda b: (0, 0, 0)),
            pl.BlockSpec((D, 1), lambda b: (0, 0)),
        ],
        out_specs=pl.BlockSpec((1, D, HQ * WQ), lambda b: (b, 0, 0)),
        out_shape=jax.ShapeDtypeStruct((B, D, HQ * WQ), f),
    )(TE, TO, w2, b2)

    z_flat = z_e.transpose(1, 0, 2).reshape(D, N)
    cbT = codebook.T
    idx3, zq, loss = pl.pallas_call(
        _vq_body,
        grid=(NCHUNKS,),
        in_specs=[
            pl.BlockSpec((D, CHUNK), lambda i: (0, i)),
            pl.BlockSpec((NC, D), lambda i: (0, 0)),
            pl.BlockSpec((D, NC), lambda i: (0, 0)),
        ],
        out_specs=[
            pl.BlockSpec((1, 1, CHUNK), lambda i: (i, 0, 0)),
            pl.BlockSpec((D, CHUNK), lambda i: (0, i)),
            pl.BlockSpec((1, 1), lambda i: (0, 0)),
        ],
        out_shape=[
            jax.ShapeDtypeStruct((NCHUNKS, 1, CHUNK), jnp.int32),
            jax.ShapeDtypeStruct((D, N), f),
            jax.ShapeDtypeStruct((1, 1), f),
        ],
    )(z_flat, codebook, cbT)
    indices = idx3.reshape(N)
    commit_loss = loss[0, 0]

    zq_b = zq.reshape(D, B, HQ * WQ).transpose(1, 0, 2)
    wd = jnp.stack([dec_w1[:, :, kh, kw].T
                    for kh in range(4) for kw in range(4)])
    bd1 = dec_b1.reshape(32, 1)
    d_par = pl.pallas_call(
        _dec1_body,
        grid=(B,),
        in_specs=[
            pl.BlockSpec((1, D, HQ * WQ), lambda b: (b, 0, 0)),
            pl.BlockSpec((16, 32, D), lambda b: (0, 0, 0)),
            pl.BlockSpec((32, 1), lambda b: (0, 0)),
        ],
        out_specs=pl.BlockSpec((1, 4, 32, HQ * WQ), lambda b: (b, 0, 0, 0)),
        out_shape=jax.ShapeDtypeStruct((B, 4, 32, HQ * WQ), f),
    )(zq_b, wd, bd1)
    d_full = d_par.reshape(B, 2, 2, 32, HQ, WQ).transpose(0, 3, 4, 1, 5, 2)
    d_full = d_full.reshape(B, 32, H1 * W1)

    w5 = jnp.stack([dec_w2[:, 0, kh, kw]
                    for kh in range(4) for kw in range(4)])
    bd2 = dec_b2.reshape(1, 1)
    xh_par = pl.pallas_call(
        _dec2_body,
        grid=(B,),
        in_specs=[
            pl.BlockSpec((1, 32, H1 * W1), lambda b: (b, 0, 0)),
            pl.BlockSpec((16, 32), lambda b: (0, 0)),
            pl.BlockSpec((1, 1), lambda b: (0, 0)),
        ],
        out_specs=pl.BlockSpec((1, 4, H1, W1), lambda b: (b, 0, 0, 0)),
        out_shape=jax.ShapeDtypeStruct((B, 4, H1, W1), f),
    )(d_full, w5, bd2)
    x_hat = xh_par.reshape(B, 2, 2, H1, W1).transpose(0, 3, 1, 4, 2)
    x_hat = x_hat.reshape(B, 1, H, W)

    return (x_hat, indices, commit_loss)
